# baseline (speedup 1.0000x reference)
import functools

import jax
import jax.numpy as jnp
from jax import lax
from jax.experimental import pallas as pl
from jax.experimental.pallas import tpu as pltpu
from jax.experimental.pallas import tpu_sc as plsc

NC = 2
NS = 16
NW = NC * NS
K = 128
NBUF = 2


def _fill_zeros_1d(ref):
    for i in range(ref.shape[0] // 16):
        ref[pl.ds(i * 16, 16)] = jnp.zeros((16,), jnp.float32)


def _fill_ones_1d(ref):
    for i in range(ref.shape[0] // 16):
        ref[pl.ds(i * 16, 16)] = jnp.ones((16,), jnp.float32)


def _deg_body(col_hbm, out_hbm, idx_v, ones_v, zero_v, sem, deg_sh):
    c = lax.axis_index("c")
    s = lax.axis_index("s")
    w = c * NS + s
    n_chunk = idx_v.shape[0]
    nacc = deg_sh.shape[0]
    per_tile = nacc // NS
    zr = zero_v.shape[0]

    _fill_ones_1d(ones_v)
    _fill_zeros_1d(zero_v)
    for i in range(per_tile // zr):
        pltpu.sync_copy(zero_v, deg_sh.at[pl.ds(s * per_tile + i * zr, zr)])
    plsc.subcore_barrier()

    pltpu.sync_copy(col_hbm.at[w], idx_v)

    def body(j, _):
        pltpu.async_copy(ones_v, deg_sh.at[idx_v.at[j]], sem, add=True)
        return 0

    lax.fori_loop(0, n_chunk, body, 0)

    def drain(j, _):
        pltpu.make_async_copy(col_hbm.at[w, 0], idx_v.at[0], sem).wait()
        return 0

    lax.fori_loop(0, n_chunk, drain, 0)
    plsc.subcore_barrier()
    pltpu.sync_copy(deg_sh.at[pl.ds(s * per_tile, per_tile)],
                    out_hbm.at[c, pl.ds(s * per_tile, per_tile)])


def _make_deg_kernel(n_chunk, nacc):
    return functools.partial(
        pl.kernel,
        out_type=jax.ShapeDtypeStruct((NC, nacc), jnp.float32),
        mesh=plsc.VectorSubcoreMesh(core_axis_name="c", subcore_axis_name="s"),
        scratch_types=[
            pltpu.VMEM((n_chunk, K), jnp.int32),
            pltpu.VMEM((K,), jnp.float32),
            pltpu.VMEM((64,), jnp.float32),
            pltpu.SemaphoreType.DMA,
            pltpu.VMEM_SHARED((nacc,), jnp.float32),
        ],
    )(_deg_body)


def _spmm_body(col_hbm, row_hbm, dinvx_hbm, out_hbm,
               colb, rowb, rows_v, isem, gsem0, gsem1, ssem0, ssem1, acc_sh):
    c = lax.axis_index("c")
    s = lax.axis_index("s")
    w = c * NS + s
    g_num = col_hbm.shape[1]
    nacc = acc_sh.shape[0]
    per_tile = nacc // NS
    gsems = [gsem0, gsem1]
    ssems = [ssem0, ssem1]
    d = rows_v.shape[2]

    def zfill(i, _):
        for k in range(d // 16):
            rows_v[0, i, pl.ds(k * 16, 16)] = jnp.zeros((16,), jnp.float32)
        return 0

    lax.fori_loop(0, K, zfill, 0)
    for i in range(per_tile // K):
        pltpu.sync_copy(rows_v.at[0], acc_sh.at[pl.ds(s * per_tile + i * K, K)])
    plsc.subcore_barrier()

    pltpu.sync_copy(col_hbm.at[w, 0], colb.at[0])
    pltpu.sync_copy(row_hbm.at[w, 0], rowb.at[0])
    if g_num > 1:
        pltpu.async_copy(col_hbm.at[w, 1], colb.at[1], isem)
        pltpu.async_copy(row_hbm.at[w, 1], rowb.at[1], isem)
    for b in range(NBUF):
        pltpu.async_copy(dinvx_hbm.at[colb.at[0, b]], rows_v.at[b], gsems[b])

    def group(g, _):
        p = lax.rem(g, 2)

        @pl.when(g + 1 < g_num)
        def _():
            pltpu.make_async_copy(col_hbm.at[w, 0], colb.at[0], isem).wait()
            pltpu.make_async_copy(row_hbm.at[w, 0], rowb.at[0], isem).wait()

        for b in range(NBUF):
            pltpu.make_async_copy(
                dinvx_hbm.at[pl.ds(0, K)], rows_v.at[b], gsems[b]).wait()
            pltpu.async_copy(rows_v.at[b], acc_sh.at[rowb.at[p, b]],
                             ssems[b], add=True)

        for b in range(NBUF):
            pltpu.make_async_copy(
                rows_v.at[b], acc_sh.at[pl.ds(0, K)], ssems[b]).wait()

            @pl.when(g + 1 < g_num)
            def _():
                pltpu.async_copy(dinvx_hbm.at[colb.at[1 - p, b]],
                                 rows_v.at[b], gsems[b])

        @pl.when(g + 2 < g_num)
        def _():
            pltpu.async_copy(col_hbm.at[w, g + 2], colb.at[p], isem)
            pltpu.async_copy(row_hbm.at[w, g + 2], rowb.at[p], isem)

        return 0

    lax.fori_loop(0, g_num, group, 0)
    plsc.subcore_barrier()
    pltpu.sync_copy(acc_sh.at[pl.ds(s * per_tile, per_tile)],
                    out_hbm.at[c, pl.ds(s * per_tile, per_tile)])


def _make_spmm_kernel(g_num, nacc, d):
    return functools.partial(
        pl.kernel,
        out_type=jax.ShapeDtypeStruct((NC, nacc, d), jnp.float32),
        mesh=plsc.VectorSubcoreMesh(core_axis_name="c", subcore_axis_name="s"),
        scratch_types=[
            pltpu.VMEM((2, NBUF, K), jnp.int32),
            pltpu.VMEM((2, NBUF, K), jnp.int32),
            pltpu.VMEM((NBUF, K, d), jnp.float32),
            pltpu.SemaphoreType.DMA,
            pltpu.SemaphoreType.DMA,
            pltpu.SemaphoreType.DMA,
            pltpu.SemaphoreType.DMA,
            pltpu.SemaphoreType.DMA,
            pltpu.VMEM_SHARED((nacc, d), jnp.float32),
        ],
    )(_spmm_body)


def _dense_body(x_ref, w_ref, degp_ref, b_ref, dinvx_ref, base_ref):
    sup = jnp.dot(x_ref[...], w_ref[...], preferred_element_type=jnp.float32)
    d = degp_ref[0, :] + degp_ref[1, :]
    dinvx_ref[...] = sup * (1.0 / d)[:, None]
    base_ref[...] = sup * (5.0 / 6.0) + b_ref[...][None, :]


def _combine_body(base_ref, p_ref, out_ref):
    out_ref[...] = base_ref[...] + (p_ref[0] + p_ref[1]) * (1.0 / 6.0)


def kernel(input, edge_index, W, b):
    n, d_feat = input.shape
    d_out = W.shape[1]
    e = edge_index.shape[1]

    gk = NW * K * NBUF
    g_num = -(-e // gk)
    ep = gk * g_num
    nacc = -(-(n + 1) // (NS * K)) * (NS * K)

    row = edge_index[0]
    col = edge_index[1]
    pad = ep - e
    if pad:
        fill = n + jnp.arange(pad, dtype=jnp.int32) % (nacc - n)
        row = jnp.concatenate([row, fill])
        col = jnp.concatenate([col, fill])
    col4 = col.reshape(NW, g_num, NBUF, K)
    row4 = row.reshape(NW, g_num, NBUF, K)

    degp = _make_deg_kernel(g_num * NBUF, nacc)(
        col4.reshape(NW, g_num * NBUF, K))

    bm = 1024
    grid = nacc // bm
    dinvx, base = pl.pallas_call(
        _dense_body,
        grid=(grid,),
        in_specs=[
            pl.BlockSpec((bm, d_feat), lambda j: (j, 0)),
            pl.BlockSpec((d_feat, d_out), lambda j: (0, 0)),
            pl.BlockSpec((NC, bm), lambda j: (0, j)),
            pl.BlockSpec((d_out,), lambda j: (0,)),
        ],
        out_specs=[
            pl.BlockSpec((bm, d_out), lambda j: (j, 0)),
            pl.BlockSpec((bm, d_out), lambda j: (j, 0)),
        ],
        out_shape=[
            jax.ShapeDtypeStruct((nacc, d_out), jnp.float32),
            jax.ShapeDtypeStruct((nacc, d_out), jnp.float32),
        ],
    )(input, W, degp, b)

    partials = _make_spmm_kernel(g_num, nacc, d_out)(col4, row4, dinvx)

    bm2 = 1000
    out = pl.pallas_call(
        _combine_body,
        grid=(n // bm2,),
        in_specs=[
            pl.BlockSpec((bm2, d_out), lambda j: (j, 0)),
            pl.BlockSpec((NC, bm2, d_out), lambda j: (0, j, 0)),
        ],
        out_specs=pl.BlockSpec((bm2, d_out), lambda j: (j, 0)),
        out_shape=jax.ShapeDtypeStruct((n, d_out), jnp.float32),
    )(base, partials)
    return out


# speedup vs baseline: 1.2082x; 1.2082x over previous
import functools

import jax
import jax.numpy as jnp
from jax import lax
from jax.experimental import pallas as pl
from jax.experimental.pallas import tpu as pltpu
from jax.experimental.pallas import tpu_sc as plsc

NC = 2
NS = 16
NW = NC * NS
K = 128
NBUF = 2


def _fill_zeros_1d(ref):
    for i in range(ref.shape[0] // 16):
        ref[pl.ds(i * 16, 16)] = jnp.zeros((16,), jnp.float32)


def _fill_ones_1d(ref):
    for i in range(ref.shape[0] // 16):
        ref[pl.ds(i * 16, 16)] = jnp.ones((16,), jnp.float32)


def _deg_body(col_hbm, out_hbm, idx_v, ones_v, zero_v, sem, deg_sh):
    c = lax.axis_index("c")
    s = lax.axis_index("s")
    w = c * NS + s
    n_chunk = idx_v.shape[0]
    nacc = deg_sh.shape[0]
    per_tile = nacc // NS
    zr = zero_v.shape[0]

    _fill_ones_1d(ones_v)
    _fill_zeros_1d(zero_v)
    for i in range(per_tile // zr):
        pltpu.sync_copy(zero_v, deg_sh.at[pl.ds(s * per_tile + i * zr, zr)])
    plsc.subcore_barrier()

    pltpu.sync_copy(col_hbm.at[w], idx_v)

    def body(j, _):
        pltpu.async_copy(ones_v, deg_sh.at[idx_v.at[j]], sem, add=True)
        return 0

    lax.fori_loop(0, n_chunk, body, 0)

    def drain(j, _):
        pltpu.make_async_copy(col_hbm.at[w, 0], idx_v.at[0], sem).wait()
        return 0

    lax.fori_loop(0, n_chunk, drain, 0)
    plsc.subcore_barrier()
    pltpu.sync_copy(deg_sh.at[pl.ds(s * per_tile, per_tile)],
                    out_hbm.at[c, pl.ds(s * per_tile, per_tile)])


def _make_deg_kernel(n_chunk, nacc):
    return functools.partial(
        pl.kernel,
        out_type=jax.ShapeDtypeStruct((NC, nacc), jnp.float32),
        mesh=plsc.VectorSubcoreMesh(core_axis_name="c", subcore_axis_name="s"),
        scratch_types=[
            pltpu.VMEM((n_chunk, K), jnp.int32),
            pltpu.VMEM((K,), jnp.float32),
            pltpu.VMEM((64,), jnp.float32),
            pltpu.SemaphoreType.DMA,
            pltpu.VMEM_SHARED((nacc,), jnp.float32),
        ],
    )(_deg_body)


def _spmm_body(col_hbm, row_hbm, dinvx_hbm, out_hbm,
               colb, rowb, rows_v, isem, gsem0, gsem1, acc_sh):
    c = lax.axis_index("c")
    s = lax.axis_index("s")
    w = c * NS + s
    g_num = col_hbm.shape[1]
    nacc = acc_sh.shape[0]
    per_tile = nacc // NS
    gsems = [gsem0, gsem1]
    d = rows_v.shape[2]

    def zfill(i, _):
        for k in range(d // 16):
            rows_v[0, i, pl.ds(k * 16, 16)] = jnp.zeros((16,), jnp.float32)
        return 0

    lax.fori_loop(0, K, zfill, 0)
    for i in range(per_tile // K):
        pltpu.sync_copy(rows_v.at[0], acc_sh.at[pl.ds(s * per_tile + i * K, K)])
    plsc.subcore_barrier()

    pltpu.sync_copy(col_hbm.at[w, 0], colb.at[0])
    pltpu.sync_copy(row_hbm.at[w, 0], rowb.at[0])
    if g_num > 1:
        pltpu.async_copy(col_hbm.at[w, 1], colb.at[1], isem)
        pltpu.async_copy(row_hbm.at[w, 1], rowb.at[1], isem)
    for b in range(NBUF):
        pltpu.async_copy(dinvx_hbm.at[colb.at[0, b]], rows_v.at[b], gsems[b])

    def group(g, _):
        p = lax.rem(g, 2)

        @pl.when(g + 1 < g_num)
        def _():
            pltpu.make_async_copy(col_hbm.at[w, 0], colb.at[0], isem).wait()
            pltpu.make_async_copy(row_hbm.at[w, 0], rowb.at[0], isem).wait()

        for b in range(NBUF):
            pltpu.make_async_copy(
                dinvx_hbm.at[pl.ds(0, K)], rows_v.at[b], gsems[b]).wait()
            pltpu.sync_copy(rows_v.at[b], acc_sh.at[rowb.at[p, b]], add=True)

            @pl.when(g + 1 < g_num)
            def _():
                pltpu.async_copy(dinvx_hbm.at[colb.at[1 - p, b]],
                                 rows_v.at[b], gsems[b])

        @pl.when(g + 2 < g_num)
        def _():
            pltpu.async_copy(col_hbm.at[w, g + 2], colb.at[p], isem)
            pltpu.async_copy(row_hbm.at[w, g + 2], rowb.at[p], isem)

        return 0

    lax.fori_loop(0, g_num, group, 0)
    plsc.subcore_barrier()
    pltpu.sync_copy(acc_sh.at[pl.ds(s * per_tile, per_tile)],
                    out_hbm.at[c, pl.ds(s * per_tile, per_tile)])


def _make_spmm_kernel(g_num, nacc, d):
    return functools.partial(
        pl.kernel,
        out_type=jax.ShapeDtypeStruct((NC, nacc, d), jnp.float32),
        mesh=plsc.VectorSubcoreMesh(core_axis_name="c", subcore_axis_name="s"),
        scratch_types=[
            pltpu.VMEM((2, NBUF, K), jnp.int32),
            pltpu.VMEM((2, NBUF, K), jnp.int32),
            pltpu.VMEM((NBUF, K, d), jnp.float32),
            pltpu.SemaphoreType.DMA,
            pltpu.SemaphoreType.DMA,
            pltpu.SemaphoreType.DMA,
            pltpu.VMEM_SHARED((nacc, d), jnp.float32),
        ],
    )(_spmm_body)


def _dense_body(x_ref, w_ref, degp_ref, b_ref, dinvx_ref, base_ref):
    sup = jnp.dot(x_ref[...], w_ref[...], preferred_element_type=jnp.float32)
    d = degp_ref[0, :] + degp_ref[1, :]
    dinvx_ref[...] = sup * (1.0 / d)[:, None]
    base_ref[...] = sup * (5.0 / 6.0) + b_ref[...][None, :]


def _combine_body(base_ref, p_ref, out_ref):
    out_ref[...] = base_ref[...] + (p_ref[0] + p_ref[1]) * (1.0 / 6.0)


def kernel(input, edge_index, W, b):
    n, d_feat = input.shape
    d_out = W.shape[1]
    e = edge_index.shape[1]

    gk = NW * K * NBUF
    g_num = -(-e // gk)
    ep = gk * g_num
    nacc = -(-(n + 1) // (NS * K)) * (NS * K)

    row = edge_index[0]
    col = edge_index[1]
    pad = ep - e
    if pad:
        fill = n + jnp.arange(pad, dtype=jnp.int32) % (nacc - n)
        row = jnp.concatenate([row, fill])
        col = jnp.concatenate([col, fill])
    col4 = col.reshape(NW, g_num, NBUF, K)
    row4 = row.reshape(NW, g_num, NBUF, K)

    degp = _make_deg_kernel(g_num * NBUF, nacc)(
        col4.reshape(NW, g_num * NBUF, K))

    bm = 1024
    grid = nacc // bm
    dinvx, base = pl.pallas_call(
        _dense_body,
        grid=(grid,),
        in_specs=[
            pl.BlockSpec((bm, d_feat), lambda j: (j, 0)),
            pl.BlockSpec((d_feat, d_out), lambda j: (0, 0)),
            pl.BlockSpec((NC, bm), lambda j: (0, j)),
            pl.BlockSpec((d_out,), lambda j: (0,)),
        ],
        out_specs=[
            pl.BlockSpec((bm, d_out), lambda j: (j, 0)),
            pl.BlockSpec((bm, d_out), lambda j: (j, 0)),
        ],
        out_shape=[
            jax.ShapeDtypeStruct((nacc, d_out), jnp.float32),
            jax.ShapeDtypeStruct((nacc, d_out), jnp.float32),
        ],
    )(input, W, degp, b)

    partials = _make_spmm_kernel(g_num, nacc, d_out)(col4, row4, dinvx)

    bm2 = 1000
    out = pl.pallas_call(
        _combine_body,
        grid=(n // bm2,),
        in_specs=[
            pl.BlockSpec((bm2, d_out), lambda j: (j, 0)),
            pl.BlockSpec((NC, bm2, d_out), lambda j: (0, j, 0)),
        ],
        out_specs=pl.BlockSpec((bm2, d_out), lambda j: (j, 0)),
        out_shape=jax.ShapeDtypeStruct((n, d_out), jnp.float32),
    )(base, partials)
    return out


# speedup vs baseline: 1.2137x; 1.0045x over previous
import functools

import jax
import jax.numpy as jnp
from jax import lax
from jax.experimental import pallas as pl
from jax.experimental.pallas import tpu as pltpu
from jax.experimental.pallas import tpu_sc as plsc

NC = 2
NS = 16
NW = NC * NS
K = 128
WIN = 8


def _fill_zeros_1d(ref):
    for i in range(ref.shape[0] // 16):
        ref[pl.ds(i * 16, 16)] = jnp.zeros((16,), jnp.float32)


def _fill_ones_1d(ref):
    for i in range(ref.shape[0] // 16):
        ref[pl.ds(i * 16, 16)] = jnp.ones((16,), jnp.float32)


def _deg_body(col_hbm, out_hbm, idx_v, ones_v, zero_v, sem, deg_sh):
    c = lax.axis_index("c")
    s = lax.axis_index("s")
    w = c * NS + s
    n_chunk = idx_v.shape[0]
    nacc = deg_sh.shape[0]
    per_tile = nacc // NS
    zr = zero_v.shape[0]

    _fill_ones_1d(ones_v)
    _fill_zeros_1d(zero_v)
    for i in range(per_tile // zr):
        pltpu.sync_copy(zero_v, deg_sh.at[pl.ds(s * per_tile + i * zr, zr)])
    plsc.subcore_barrier()

    pltpu.sync_copy(col_hbm.at[w], idx_v)

    def body(j, _):
        pltpu.async_copy(ones_v, deg_sh.at[idx_v.at[j]], sem, add=True)
        return 0

    lax.fori_loop(0, n_chunk, body, 0)

    def drain(j, _):
        pltpu.make_async_copy(col_hbm.at[w, 0], idx_v.at[0], sem).wait()
        return 0

    lax.fori_loop(0, n_chunk, drain, 0)
    plsc.subcore_barrier()
    pltpu.sync_copy(deg_sh.at[pl.ds(s * per_tile, per_tile)],
                    out_hbm.at[c, pl.ds(s * per_tile, per_tile)])


def _make_deg_kernel(n_chunk, nacc):
    return functools.partial(
        pl.kernel,
        out_type=jax.ShapeDtypeStruct((NC, nacc), jnp.float32),
        mesh=plsc.VectorSubcoreMesh(core_axis_name="c", subcore_axis_name="s"),
        scratch_types=[
            pltpu.VMEM((n_chunk, K), jnp.int32),
            pltpu.VMEM((K,), jnp.float32),
            pltpu.VMEM((K,), jnp.float32),
            pltpu.SemaphoreType.DMA,
            pltpu.VMEM_SHARED((nacc,), jnp.float32),
        ],
    )(_deg_body)


def _spmm_body(col_hbm, row_hbm, dinvx_hbm, out_hbm,
               colb, rowb, rows_v, isem, gsem0, gsem1, acc_sh):
    c = lax.axis_index("c")
    s = lax.axis_index("s")
    w = c * NS + s
    g_num = col_hbm.shape[1] // WIN
    nacc = acc_sh.shape[0]
    per_tile = nacc // NS
    gsems = [gsem0, gsem1]
    d = rows_v.shape[2]

    def zfill(i, _):
        for k in range(d // 16):
            rows_v[0, i, pl.ds(k * 16, 16)] = jnp.zeros((16,), jnp.float32)
        return 0

    lax.fori_loop(0, K, zfill, 0)
    for i in range(per_tile // K):
        pltpu.sync_copy(rows_v.at[0], acc_sh.at[pl.ds(s * per_tile + i * K, K)])
    plsc.subcore_barrier()

    pltpu.sync_copy(col_hbm.at[w, pl.ds(0, WIN)], colb.at[0])
    pltpu.sync_copy(row_hbm.at[w, pl.ds(0, WIN)], rowb.at[0])
    if g_num > 1:
        pltpu.async_copy(col_hbm.at[w, pl.ds(WIN, WIN)], colb.at[1], isem)
        pltpu.async_copy(row_hbm.at[w, pl.ds(WIN, WIN)], rowb.at[1], isem)
    for b in range(2):
        pltpu.async_copy(dinvx_hbm.at[colb.at[0, b]], rows_v.at[b], gsems[b])

    def win(g, _):
        p = lax.rem(g, 2)

        @pl.when(g + 1 < g_num)
        def _():
            pltpu.make_async_copy(col_hbm.at[w, pl.ds(0, WIN)], colb.at[0],
                                  isem).wait()
            pltpu.make_async_copy(row_hbm.at[w, pl.ds(0, WIN)], rowb.at[0],
                                  isem).wait()

        for b in range(WIN):
            rb = b % 2
            pltpu.make_async_copy(
                dinvx_hbm.at[pl.ds(0, K)], rows_v.at[rb], gsems[rb]).wait()
            pltpu.sync_copy(rows_v.at[rb], acc_sh.at[rowb.at[p, b]], add=True)
            if b < WIN - 2:
                pltpu.async_copy(dinvx_hbm.at[colb.at[p, b + 2]],
                                 rows_v.at[rb], gsems[rb])
            else:
                @pl.when(g + 1 < g_num)
                def _():
                    pltpu.async_copy(dinvx_hbm.at[colb.at[1 - p, b - (WIN - 2)]],
                                     rows_v.at[rb], gsems[rb])

        @pl.when(g + 2 < g_num)
        def _():
            pltpu.async_copy(col_hbm.at[w, pl.ds((g + 2) * WIN, WIN)],
                             colb.at[p], isem)
            pltpu.async_copy(row_hbm.at[w, pl.ds((g + 2) * WIN, WIN)],
                             rowb.at[p], isem)

        return 0

    lax.fori_loop(0, g_num, win, 0)
    plsc.subcore_barrier()
    pltpu.sync_copy(acc_sh.at[pl.ds(s * per_tile, per_tile)],
                    out_hbm.at[c, pl.ds(s * per_tile, per_tile)])


def _make_spmm_kernel(n_chunk, nacc, d):
    return functools.partial(
        pl.kernel,
        out_type=jax.ShapeDtypeStruct((NC, nacc, d), jnp.float32),
        mesh=plsc.VectorSubcoreMesh(core_axis_name="c", subcore_axis_name="s"),
        scratch_types=[
            pltpu.VMEM((2, WIN, K), jnp.int32),
            pltpu.VMEM((2, WIN, K), jnp.int32),
            pltpu.VMEM((2, K, d), jnp.float32),
            pltpu.SemaphoreType.DMA,
            pltpu.SemaphoreType.DMA,
            pltpu.SemaphoreType.DMA,
            pltpu.VMEM_SHARED((nacc, d), jnp.float32),
        ],
    )(_spmm_body)


def _dense_body(x_ref, w_ref, degp_ref, b_ref, dinvx_ref, base_ref):
    sup = jnp.dot(x_ref[...], w_ref[...], preferred_element_type=jnp.float32)
    d = degp_ref[0, :] + degp_ref[1, :]
    dinvx_ref[...] = sup * (1.0 / d)[:, None]
    base_ref[...] = sup * (5.0 / 6.0) + b_ref[...][None, :]


def _combine_body(base_ref, p_ref, out_ref):
    out_ref[...] = base_ref[...] + (p_ref[0] + p_ref[1]) * (1.0 / 6.0)


def kernel(input, edge_index, W, b):
    n, d_feat = input.shape
    d_out = W.shape[1]
    e = edge_index.shape[1]

    ch = -(-e // (NW * K))
    ch = -(-ch // WIN) * WIN
    ep = NW * ch * K
    nacc = -(-(n + 1) // (NS * K)) * (NS * K)

    row = edge_index[0]
    col = edge_index[1]
    pad = ep - e
    if pad:
        fill = n + jnp.arange(pad, dtype=jnp.int32) % (nacc - n)
        row = jnp.concatenate([row, fill])
        col = jnp.concatenate([col, fill])
    col3 = col.reshape(NW, ch, K)
    row3 = row.reshape(NW, ch, K)

    degp = _make_deg_kernel(ch, nacc)(col3)

    bm = 1024
    grid = nacc // bm
    dinvx, base = pl.pallas_call(
        _dense_body,
        grid=(grid,),
        in_specs=[
            pl.BlockSpec((bm, d_feat), lambda j: (j, 0)),
            pl.BlockSpec((d_feat, d_out), lambda j: (0, 0)),
            pl.BlockSpec((NC, bm), lambda j: (0, j)),
            pl.BlockSpec((d_out,), lambda j: (0,)),
        ],
        out_specs=[
            pl.BlockSpec((bm, d_out), lambda j: (j, 0)),
            pl.BlockSpec((bm, d_out), lambda j: (j, 0)),
        ],
        out_shape=[
            jax.ShapeDtypeStruct((nacc, d_out), jnp.float32),
            jax.ShapeDtypeStruct((nacc, d_out), jnp.float32),
        ],
    )(input, W, degp, b)

    partials = _make_spmm_kernel(ch, nacc, d_out)(col3, row3, dinvx)

    bm2 = 1000
    out = pl.pallas_call(
        _combine_body,
        grid=(n // bm2,),
        in_specs=[
            pl.BlockSpec((bm2, d_out), lambda j: (j, 0)),
            pl.BlockSpec((NC, bm2, d_out), lambda j: (0, j, 0)),
        ],
        out_specs=pl.BlockSpec((bm2, d_out), lambda j: (j, 0)),
        out_shape=jax.ShapeDtypeStruct((n, d_out), jnp.float32),
    )(base, partials)
    return out


# speedup vs baseline: 1.2727x; 1.0486x over previous
import functools

import jax
import jax.numpy as jnp
from jax import lax
from jax.experimental import pallas as pl
from jax.experimental.pallas import tpu as pltpu
from jax.experimental.pallas import tpu_sc as plsc

NC = 2
NS = 16
NW = NC * NS
K = 128
WIN = 8


def _fill_zeros_1d(ref):
    for i in range(ref.shape[0] // 16):
        ref[pl.ds(i * 16, 16)] = jnp.zeros((16,), jnp.float32)


def _fill_ones_1d(ref):
    for i in range(ref.shape[0] // 16):
        ref[pl.ds(i * 16, 16)] = jnp.ones((16,), jnp.float32)


def _deg_body(ei_hbm, out_hbm, idx_v, ones_v, zero_v, sem, deg_sh):
    c = lax.axis_index("c")
    s = lax.axis_index("s")
    w = c * NS + s
    n_chunk = idx_v.shape[0]
    nacc = deg_sh.shape[0]
    per_tile = nacc // NS
    zr = zero_v.shape[0]

    _fill_ones_1d(ones_v)
    _fill_zeros_1d(zero_v)
    for i in range(per_tile // zr):
        pltpu.sync_copy(zero_v, deg_sh.at[pl.ds(s * per_tile + i * zr, zr)])
    plsc.subcore_barrier()

    pltpu.sync_copy(ei_hbm.at[1, pl.ds(w * n_chunk, n_chunk)], idx_v)

    def body(j, _):
        pltpu.async_copy(ones_v, deg_sh.at[idx_v.at[j]], sem, add=True)
        return 0

    lax.fori_loop(0, n_chunk, body, 0)

    def drain(j, _):
        pltpu.make_async_copy(ei_hbm.at[1, 0], idx_v.at[0], sem).wait()
        return 0

    lax.fori_loop(0, n_chunk, drain, 0)
    plsc.subcore_barrier()
    pltpu.sync_copy(deg_sh.at[pl.ds(s * per_tile, per_tile)],
                    out_hbm.at[c, pl.ds(s * per_tile, per_tile)])


def _make_deg_kernel(n_chunk, nacc):
    return functools.partial(
        pl.kernel,
        out_type=jax.ShapeDtypeStruct((NC, nacc), jnp.float32),
        mesh=plsc.VectorSubcoreMesh(core_axis_name="c", subcore_axis_name="s"),
        scratch_types=[
            pltpu.VMEM((n_chunk, K), jnp.int32),
            pltpu.VMEM((K,), jnp.float32),
            pltpu.VMEM((K,), jnp.float32),
            pltpu.SemaphoreType.DMA,
            pltpu.VMEM_SHARED((nacc,), jnp.float32),
        ],
    )(_deg_body)


def _spmm_body(ei_hbm, dinvx_hbm, out_hbm,
               colb, rowb, rows_v, isem, gsem0, gsem1, acc_sh):
    c = lax.axis_index("c")
    s = lax.axis_index("s")
    w = c * NS + s
    n_chunk = ei_hbm.shape[1] // NW
    g_num = n_chunk // WIN
    cb = w * n_chunk
    nacc = acc_sh.shape[0]
    per_tile = nacc // NS
    gsems = [gsem0, gsem1]
    d = rows_v.shape[2]

    def zfill(i, _):
        for k in range(d // 16):
            rows_v[0, i, pl.ds(k * 16, 16)] = jnp.zeros((16,), jnp.float32)
        return 0

    lax.fori_loop(0, K, zfill, 0)
    for i in range(per_tile // K):
        pltpu.sync_copy(rows_v.at[0], acc_sh.at[pl.ds(s * per_tile + i * K, K)])
    plsc.subcore_barrier()

    pltpu.sync_copy(ei_hbm.at[1, pl.ds(cb, WIN)], colb.at[0])
    pltpu.sync_copy(ei_hbm.at[0, pl.ds(cb, WIN)], rowb.at[0])
    if g_num > 1:
        pltpu.async_copy(ei_hbm.at[1, pl.ds(cb + WIN, WIN)], colb.at[1], isem)
        pltpu.async_copy(ei_hbm.at[0, pl.ds(cb + WIN, WIN)], rowb.at[1], isem)
    for b in range(2):
        pltpu.async_copy(dinvx_hbm.at[colb.at[0, b]], rows_v.at[b], gsems[b])

    def win(g, _):
        p = lax.rem(g, 2)

        @pl.when(g + 1 < g_num)
        def _():
            pltpu.make_async_copy(ei_hbm.at[1, pl.ds(0, WIN)], colb.at[0],
                                  isem).wait()
            pltpu.make_async_copy(ei_hbm.at[0, pl.ds(0, WIN)], rowb.at[0],
                                  isem).wait()

        for b in range(WIN):
            rb = b % 2
            pltpu.make_async_copy(
                dinvx_hbm.at[pl.ds(0, K)], rows_v.at[rb], gsems[rb]).wait()
            pltpu.sync_copy(rows_v.at[rb], acc_sh.at[rowb.at[p, b]], add=True)
            if b < WIN - 2:
                pltpu.async_copy(dinvx_hbm.at[colb.at[p, b + 2]],
                                 rows_v.at[rb], gsems[rb])
            else:
                @pl.when(g + 1 < g_num)
                def _():
                    pltpu.async_copy(dinvx_hbm.at[colb.at[1 - p, b - (WIN - 2)]],
                                     rows_v.at[rb], gsems[rb])

        @pl.when(g + 2 < g_num)
        def _():
            pltpu.async_copy(ei_hbm.at[1, pl.ds(cb + (g + 2) * WIN, WIN)],
                             colb.at[p], isem)
            pltpu.async_copy(ei_hbm.at[0, pl.ds(cb + (g + 2) * WIN, WIN)],
                             rowb.at[p], isem)

        return 0

    lax.fori_loop(0, g_num, win, 0)
    plsc.subcore_barrier()
    pltpu.sync_copy(acc_sh.at[pl.ds(s * per_tile, per_tile)],
                    out_hbm.at[c, pl.ds(s * per_tile, per_tile)])


def _make_spmm_kernel(n_chunk, nacc, d):
    return functools.partial(
        pl.kernel,
        out_type=jax.ShapeDtypeStruct((NC, nacc, d), jnp.float32),
        mesh=plsc.VectorSubcoreMesh(core_axis_name="c", subcore_axis_name="s"),
        scratch_types=[
            pltpu.VMEM((2, WIN, K), jnp.int32),
            pltpu.VMEM((2, WIN, K), jnp.int32),
            pltpu.VMEM((2, K, d), jnp.float32),
            pltpu.SemaphoreType.DMA,
            pltpu.SemaphoreType.DMA,
            pltpu.SemaphoreType.DMA,
            pltpu.VMEM_SHARED((nacc, d), jnp.float32),
        ],
    )(_spmm_body)


def _dense_body(x_ref, w_ref, degp_ref, b_ref, dinvx_ref, base_ref):
    sup = jnp.dot(x_ref[...], w_ref[...], preferred_element_type=jnp.float32)
    d = degp_ref[0, :] + degp_ref[1, :]
    dinvx_ref[...] = sup * (1.0 / d)[:, None]
    base_ref[...] = sup * (5.0 / 6.0) + b_ref[...][None, :]


def _combine_body(base_ref, p_ref, out_ref):
    out_ref[...] = base_ref[...] + (p_ref[0] + p_ref[1]) * (1.0 / 6.0)


def kernel(input, edge_index, W, b):
    n, d_feat = input.shape
    d_out = W.shape[1]
    e = edge_index.shape[1]

    ch = -(-e // (NW * K))
    ch = -(-ch // WIN) * WIN
    ep = NW * ch * K
    nacc = -(-(n + 1) // (NS * K)) * (NS * K)

    ei = edge_index
    pad = ep - e
    if pad:
        fill = n + jnp.arange(pad, dtype=jnp.int32) % (nacc - n)
        ei = jnp.concatenate([ei, jnp.stack([fill, fill])], axis=1)
    ei3 = ei.reshape(2, NW * ch, K)

    degp = _make_deg_kernel(ch, nacc)(ei3)

    bm = 1024
    grid = nacc // bm
    dinvx, base = pl.pallas_call(
        _dense_body,
        grid=(grid,),
        in_specs=[
            pl.BlockSpec((bm, d_feat), lambda j: (j, 0)),
            pl.BlockSpec((d_feat, d_out), lambda j: (0, 0)),
            pl.BlockSpec((NC, bm), lambda j: (0, j)),
            pl.BlockSpec((d_out,), lambda j: (0,)),
        ],
        out_specs=[
            pl.BlockSpec((bm, d_out), lambda j: (j, 0)),
            pl.BlockSpec((bm, d_out), lambda j: (j, 0)),
        ],
        out_shape=[
            jax.ShapeDtypeStruct((nacc, d_out), jnp.float32),
            jax.ShapeDtypeStruct((nacc, d_out), jnp.float32),
        ],
    )(input, W, degp, b)

    partials = _make_spmm_kernel(ch, nacc, d_out)(ei3, dinvx)

    bm2 = 1000
    out = pl.pallas_call(
        _combine_body,
        grid=(n // bm2,),
        in_specs=[
            pl.BlockSpec((bm2, d_out), lambda j: (j, 0)),
            pl.BlockSpec((NC, bm2, d_out), lambda j: (0, j, 0)),
        ],
        out_specs=pl.BlockSpec((bm2, d_out), lambda j: (j, 0)),
        out_shape=jax.ShapeDtypeStruct((n, d_out), jnp.float32),
    )(base, partials)
    return out


# speedup vs baseline: 1.3144x; 1.0327x over previous
import functools

import jax
import jax.numpy as jnp
from jax import lax
from jax.experimental import pallas as pl
from jax.experimental.pallas import tpu as pltpu
from jax.experimental.pallas import tpu_sc as plsc

NC = 2
NS = 16
NW = NC * NS
K = 128
WIN = 8


def _fill_zeros_1d(ref):
    for i in range(ref.shape[0] // 16):
        ref[pl.ds(i * 16, 16)] = jnp.zeros((16,), jnp.float32)


def _fill_ones_1d(ref):
    for i in range(ref.shape[0] // 16):
        ref[pl.ds(i * 16, 16)] = jnp.ones((16,), jnp.float32)


def _deg_body(ei_hbm, out_hbm, idx_v, ones_v, zero_v, sem, deg_sh):
    c = lax.axis_index("c")
    s = lax.axis_index("s")
    w = c * NS + s
    n_chunk = idx_v.shape[0]
    nacc = deg_sh.shape[0]
    per_tile = nacc // NS
    zr = zero_v.shape[0]

    _fill_ones_1d(ones_v)
    _fill_zeros_1d(zero_v)
    for i in range(per_tile // zr):
        pltpu.sync_copy(zero_v, deg_sh.at[pl.ds(s * per_tile + i * zr, zr)])
    plsc.subcore_barrier()

    pltpu.sync_copy(ei_hbm.at[1, pl.ds(w * n_chunk, n_chunk)], idx_v)

    def body(j, _):
        pltpu.async_copy(ones_v, deg_sh.at[idx_v.at[j]], sem, add=True)
        return 0

    lax.fori_loop(0, n_chunk, body, 0)

    def drain(j, _):
        pltpu.make_async_copy(ei_hbm.at[1, 0], idx_v.at[0], sem).wait()
        return 0

    lax.fori_loop(0, n_chunk, drain, 0)
    plsc.subcore_barrier()
    pltpu.sync_copy(deg_sh.at[pl.ds(s * per_tile, per_tile)],
                    out_hbm.at[c, pl.ds(s * per_tile, per_tile)])


def _make_deg_kernel(n_chunk, nacc):
    return functools.partial(
        pl.kernel,
        out_type=jax.ShapeDtypeStruct((NC, nacc), jnp.float32),
        mesh=plsc.VectorSubcoreMesh(core_axis_name="c", subcore_axis_name="s"),
        scratch_types=[
            pltpu.VMEM((n_chunk, K), jnp.int32),
            pltpu.VMEM((K,), jnp.float32),
            pltpu.VMEM((K,), jnp.float32),
            pltpu.SemaphoreType.DMA,
            pltpu.VMEM_SHARED((nacc,), jnp.float32),
        ],
    )(_deg_body)


def _spmm_body(ei_hbm, dinvx_hbm, out_hbm,
               colb, rowb, rows_v, isem, gsem0, gsem1, acc_sh):
    c = lax.axis_index("c")
    s = lax.axis_index("s")
    w = c * NS + s
    n_chunk = ei_hbm.shape[1] // NW
    g_num = n_chunk // WIN
    cb = w * n_chunk
    nacc = acc_sh.shape[0]
    per_tile = nacc // NS
    gsems = [gsem0, gsem1]
    d = rows_v.shape[2]

    def zfill(i, _):
        for k in range(d // 16):
            rows_v[0, i, pl.ds(k * 16, 16)] = jnp.zeros((16,), jnp.float32)
        return 0

    lax.fori_loop(0, K, zfill, 0)
    for i in range(per_tile // K):
        pltpu.sync_copy(rows_v.at[0], acc_sh.at[pl.ds(s * per_tile + i * K, K)])
    plsc.subcore_barrier()

    pltpu.sync_copy(ei_hbm.at[1, pl.ds(cb, WIN)], colb.at[0])
    pltpu.sync_copy(ei_hbm.at[0, pl.ds(cb, WIN)], rowb.at[0])
    if g_num > 1:
        pltpu.async_copy(ei_hbm.at[1, pl.ds(cb + WIN, WIN)], colb.at[1], isem)
        pltpu.async_copy(ei_hbm.at[0, pl.ds(cb + WIN, WIN)], rowb.at[1], isem)
    for b in range(2):
        pltpu.async_copy(dinvx_hbm.at[colb.at[0, b]], rows_v.at[b], gsems[b])

    def win(g, _):
        p = lax.rem(g, 2)

        @pl.when(g + 1 < g_num)
        def _():
            pltpu.make_async_copy(ei_hbm.at[1, pl.ds(0, WIN)], colb.at[0],
                                  isem).wait()
            pltpu.make_async_copy(ei_hbm.at[0, pl.ds(0, WIN)], rowb.at[0],
                                  isem).wait()

        for b in range(WIN):
            rb = b % 2
            pltpu.make_async_copy(
                dinvx_hbm.at[pl.ds(0, K)], rows_v.at[rb], gsems[rb]).wait()
            pltpu.sync_copy(rows_v.at[rb], acc_sh.at[rowb.at[p, b]], add=True)
            if b < WIN - 2:
                pltpu.async_copy(dinvx_hbm.at[colb.at[p, b + 2]],
                                 rows_v.at[rb], gsems[rb])
            else:
                @pl.when(g + 1 < g_num)
                def _():
                    pltpu.async_copy(dinvx_hbm.at[colb.at[1 - p, b - (WIN - 2)]],
                                     rows_v.at[rb], gsems[rb])

        @pl.when(g + 2 < g_num)
        def _():
            pltpu.async_copy(ei_hbm.at[1, pl.ds(cb + (g + 2) * WIN, WIN)],
                             colb.at[p], isem)
            pltpu.async_copy(ei_hbm.at[0, pl.ds(cb + (g + 2) * WIN, WIN)],
                             rowb.at[p], isem)

        return 0

    lax.fori_loop(0, g_num, win, 0)
    plsc.subcore_barrier()
    pltpu.sync_copy(acc_sh.at[pl.ds(s * per_tile, per_tile)],
                    out_hbm.at[c, pl.ds(s * per_tile, per_tile)])


def _make_spmm_kernel(n_chunk, nacc, d):
    return functools.partial(
        pl.kernel,
        out_type=jax.ShapeDtypeStruct((NC, nacc, d), jnp.float32),
        mesh=plsc.VectorSubcoreMesh(core_axis_name="c", subcore_axis_name="s"),
        scratch_types=[
            pltpu.VMEM((2, WIN, K), jnp.int32),
            pltpu.VMEM((2, WIN, K), jnp.int32),
            pltpu.VMEM((2, K, d), jnp.float32),
            pltpu.SemaphoreType.DMA,
            pltpu.SemaphoreType.DMA,
            pltpu.SemaphoreType.DMA,
            pltpu.VMEM_SHARED((nacc, d), jnp.float32),
        ],
    )(_spmm_body)


def _dense_body(x_ref, w_ref, degp_ref, b_ref, dinvx_ref, base_ref):
    sup = jnp.dot(x_ref[...], w_ref[...], preferred_element_type=jnp.float32)
    d = degp_ref[0, :] + degp_ref[1, :]
    dinvx_ref[...] = sup * (1.0 / d)[:, None]
    base_ref[...] = sup * (5.0 / 6.0) + b_ref[...][None, :]


def _combine_body(base_ref, p_ref, out_ref):
    out_ref[...] = base_ref[...] + (p_ref[0] + p_ref[1]) * (1.0 / 6.0)


def kernel(input, edge_index, W, b):
    n, d_feat = input.shape
    d_out = W.shape[1]
    e = edge_index.shape[1]

    ch = -(-e // (NW * K))
    ch = -(-ch // WIN) * WIN
    ep = NW * ch * K
    nacc = -(-(n + 1) // (NS * K)) * (NS * K)

    ei = edge_index
    pad = ep - e
    if pad:
        fill = n + jnp.arange(pad, dtype=jnp.int32) % (nacc - n)
        ei = jnp.concatenate([ei, jnp.stack([fill, fill])], axis=1)
    ei3 = ei.reshape(2, NW * ch, K)

    degp = _make_deg_kernel(ch, nacc)(ei3)

    bm = 2048
    grid = nacc // bm
    dinvx, base = pl.pallas_call(
        _dense_body,
        grid=(grid,),
        in_specs=[
            pl.BlockSpec((bm, d_feat), lambda j: (j, 0)),
            pl.BlockSpec((d_feat, d_out), lambda j: (0, 0)),
            pl.BlockSpec((NC, bm), lambda j: (0, j)),
            pl.BlockSpec((d_out,), lambda j: (0,)),
        ],
        out_specs=[
            pl.BlockSpec((bm, d_out), lambda j: (j, 0)),
            pl.BlockSpec((bm, d_out), lambda j: (j, 0)),
        ],
        out_shape=[
            jax.ShapeDtypeStruct((nacc, d_out), jnp.float32),
            jax.ShapeDtypeStruct((nacc, d_out), jnp.float32),
        ],
    )(input, W, degp, b)

    partials = _make_spmm_kernel(ch, nacc, d_out)(ei3, dinvx)

    bm2 = 2000
    out = pl.pallas_call(
        _combine_body,
        grid=(n // bm2,),
        in_specs=[
            pl.BlockSpec((bm2, d_out), lambda j: (j, 0)),
            pl.BlockSpec((NC, bm2, d_out), lambda j: (0, j, 0)),
        ],
        out_specs=pl.BlockSpec((bm2, d_out), lambda j: (j, 0)),
        out_shape=jax.ShapeDtypeStruct((n, d_out), jnp.float32),
    )(base, partials)
    return out
